# baseline (speedup 1.0000x reference)
import functools

import jax
import jax.numpy as jnp
from jax import lax
from jax.experimental import pallas as pl
from jax.experimental.pallas import tpu as pltpu
from jax.experimental.pallas import tpu_sc as plsc

_DIM = 64
_SCALE = 8.0
_COLS_BLOCK = 4096


def _transform_body(tt_ref, out_ref):
    pos = lax.broadcasted_iota(jnp.int32, (_COLS_BLOCK, _DIM), 1).astype(jnp.float32) + 1.0
    out_ref[:, : _DIM] = tt_ref[...].T * _SCALE + pos


def _transform(table_t):
    vocab = table_t.shape[1]
    return pl.pallas_call(
        _transform_body,
        grid=((vocab + _COLS_BLOCK - 1) // _COLS_BLOCK,),
        in_specs=[pl.BlockSpec((_DIM, _COLS_BLOCK), lambda i: (0, i))],
        out_specs=pl.BlockSpec((_COLS_BLOCK, 2 * _DIM), lambda i: (i, 0)),
        out_shape=jax.ShapeDtypeStruct((vocab, 2 * _DIM), jnp.float32),
    )(table_t)


_TB = 256


def _format_body(in_ref, out_ref):
    seq2 = in_ref.shape[0] // _TB
    inr = in_ref[...].reshape(_TB, seq2, 2 * _DIM)
    for l2 in range(seq2):
        st = inr[:, l2, :].T
        out_ref[0, 2 * l2] = st[:_DIM]
        out_ref[0, 2 * l2 + 1] = st[_DIM:]


def _format(lin2, batch, seq):
    return pl.pallas_call(
        _format_body,
        grid=(batch // _TB,),
        in_specs=[pl.BlockSpec((_TB * seq // 2, 2 * _DIM), lambda i: (i, 0))],
        out_specs=pl.BlockSpec((1, seq, _DIM, _TB), lambda i: (0, 0, 0, i)),
        out_shape=jax.ShapeDtypeStruct((1, seq, _DIM, batch), jnp.float32),
    )(lin2)


@functools.lru_cache(maxsize=None)
def _make_gather(batch, seq, vocab):
    info = plsc.get_sparse_core_info()
    nc, ns = info.num_cores, info.num_subcores
    nw = nc * ns
    rows = batch * seq
    rpw = rows // nw
    chunk = 64
    nchunks = rpw // chunk
    nbuf = 4
    mesh = plsc.VectorSubcoreMesh(core_axis_name="c", subcore_axis_name="s")

    @functools.partial(
        pl.kernel,
        mesh=mesh,
        compiler_params=pltpu.CompilerParams(use_tc_tiling_on_sc=False),
        out_type=jax.ShapeDtypeStruct((rows, _DIM), jnp.float32),
        scratch_types=[
            pltpu.VMEM((nchunks, chunk), jnp.int32),
        ]
        + [pltpu.VMEM((chunk, _DIM), jnp.float32)] * nbuf
        + [pltpu.SemaphoreType.DMA] * (2 * nbuf),
    )
    def k(idx_hbm, table_hbm, out_hbm, idx_v, *bufs_sems):
        bufs = bufs_sems[:nbuf]
        gs = bufs_sems[nbuf : 2 * nbuf]
        ws = bufs_sems[2 * nbuf :]
        wid = lax.axis_index("s") * nc + lax.axis_index("c")
        r0 = wid * rpw
        pltpu.sync_copy(idx_hbm.at[wid], idx_v)

        def start_gather(j, p):
            pltpu.async_copy(table_hbm.at[idx_v.at[j]], bufs[p], gs[p])

        def wait_gather(j, p):
            pltpu.make_async_copy(table_hbm.at[idx_v.at[j]], bufs[p], gs[p]).wait()

        def start_write(j, p):
            pltpu.async_copy(bufs[p], out_hbm.at[pl.ds(r0 + j * chunk, chunk)], ws[p])

        def wait_write(j, p):
            pltpu.make_async_copy(
                bufs[p], out_hbm.at[pl.ds(r0 + j * chunk, chunk)], ws[p]
            ).wait()

        start_gather(0, 0)
        start_gather(1, 1)

        def body(j4, carry):
            for p in range(nbuf):
                j = nbuf * j4 + p
                wait_gather(j, p)
                start_write(j, p)
                q = (p + 2) % nbuf

                @pl.when(j + 2 < nchunks)
                def _():
                    @pl.when(j >= 2)
                    def _():
                        wait_write(j - 2, q)

                    start_gather(j + 2, q)

            return carry

        lax.fori_loop(0, nchunks // nbuf, body, 0)
        for p in range(nbuf):
            j = nchunks - nbuf + p
            wait_write(j, j % nbuf)

    return k


def kernel(x, table):
    b, l = x.shape
    nw = plsc.get_sparse_core_info().num_cores * plsc.get_sparse_core_info().num_subcores
    idx = (x.astype(jnp.int32) * 2).reshape(nw, -1, 64)
    table2 = _transform(table.T).reshape(2 * table.shape[0], _DIM)
    out = _make_gather(b, l, 2 * table.shape[0])(idx, table2)
    t4 = _format(out.reshape(b * l // 2, 2 * _DIM), b, l)
    return t4.transpose(0, 3, 1, 2)


# speedup vs baseline: 1.0867x; 1.0867x over previous
import functools

import jax
import jax.numpy as jnp
from jax import lax
from jax.experimental import pallas as pl
from jax.experimental.pallas import tpu as pltpu
from jax.experimental.pallas import tpu_sc as plsc

_DIM = 64
_SCALE = 8.0
_COLS_BLOCK = 4096


def _transform_body(tt_ref, out_ref):
    pos = lax.broadcasted_iota(jnp.int32, (_COLS_BLOCK, _DIM), 1).astype(jnp.float32) + 1.0
    out_ref[:, : _DIM] = tt_ref[...].T * _SCALE + pos


def _transform(table_t):
    vocab = table_t.shape[1]
    return pl.pallas_call(
        _transform_body,
        grid=((vocab + _COLS_BLOCK - 1) // _COLS_BLOCK,),
        in_specs=[pl.BlockSpec((_DIM, _COLS_BLOCK), lambda i: (0, i))],
        out_specs=pl.BlockSpec((_COLS_BLOCK, 2 * _DIM), lambda i: (i, 0)),
        out_shape=jax.ShapeDtypeStruct((vocab, 2 * _DIM), jnp.float32),
    )(table_t)


_TB = 256


def _format_body(in_ref, out_ref):
    seq2 = in_ref.shape[0] // _TB
    inr = in_ref[...].reshape(_TB, seq2, 2 * _DIM)
    for l2 in range(seq2):
        st = inr[:, l2, :].T
        out_ref[0, 2 * l2] = st[:_DIM]
        out_ref[0, 2 * l2 + 1] = st[_DIM:]


def _format(lin2, batch, seq):
    return pl.pallas_call(
        _format_body,
        grid=(batch // _TB,),
        in_specs=[pl.BlockSpec((_TB * seq // 2, 2 * _DIM), lambda i: (i, 0))],
        out_specs=pl.BlockSpec((1, seq, _DIM, _TB), lambda i: (0, 0, 0, i)),
        out_shape=jax.ShapeDtypeStruct((1, seq, _DIM, batch), jnp.float32),
    )(lin2)


@functools.lru_cache(maxsize=None)
def _make_gather(batch, seq, vocab):
    info = plsc.get_sparse_core_info()
    nc, ns = info.num_cores, info.num_subcores
    nw = nc * ns
    rows = batch * seq
    rpw = rows // nw
    chunk = 128
    nchunks = rpw // chunk
    nbuf = 4
    mesh = plsc.VectorSubcoreMesh(core_axis_name="c", subcore_axis_name="s")

    @functools.partial(
        pl.kernel,
        mesh=mesh,
        compiler_params=pltpu.CompilerParams(use_tc_tiling_on_sc=False),
        out_type=jax.ShapeDtypeStruct((rows, _DIM), jnp.float32),
        scratch_types=[
            pltpu.VMEM((nchunks, chunk), jnp.int32),
        ]
        + [pltpu.VMEM((chunk, _DIM), jnp.float32)] * nbuf
        + [pltpu.SemaphoreType.DMA] * (2 * nbuf),
    )
    def k(idx_hbm, table_hbm, out_hbm, idx_v, *bufs_sems):
        bufs = bufs_sems[:nbuf]
        gs = bufs_sems[nbuf : 2 * nbuf]
        ws = bufs_sems[2 * nbuf :]
        wid = lax.axis_index("s") * nc + lax.axis_index("c")
        r0 = wid * rpw
        pltpu.sync_copy(idx_hbm.at[wid], idx_v)

        def start_gather(j, p):
            pltpu.async_copy(table_hbm.at[idx_v.at[j]], bufs[p], gs[p])

        def wait_gather(j, p):
            pltpu.make_async_copy(table_hbm.at[idx_v.at[j]], bufs[p], gs[p]).wait()

        def start_write(j, p):
            pltpu.async_copy(bufs[p], out_hbm.at[pl.ds(r0 + j * chunk, chunk)], ws[p])

        def wait_write(j, p):
            pltpu.make_async_copy(
                bufs[p], out_hbm.at[pl.ds(r0 + j * chunk, chunk)], ws[p]
            ).wait()

        start_gather(0, 0)
        start_gather(1, 1)

        def body(j4, carry):
            for p in range(nbuf):
                j = nbuf * j4 + p
                wait_gather(j, p)
                start_write(j, p)
                q = (p + 2) % nbuf

                @pl.when(j + 2 < nchunks)
                def _():
                    @pl.when(j >= 2)
                    def _():
                        wait_write(j - 2, q)

                    start_gather(j + 2, q)

            return carry

        lax.fori_loop(0, nchunks // nbuf, body, 0)
        for jt in range(nbuf * (nchunks // nbuf), nchunks):
            wait_gather(jt, jt % nbuf)
            start_write(jt, jt % nbuf)
        for p in range(nbuf):
            j = nchunks - nbuf + p
            wait_write(j, j % nbuf)

    return k


def kernel(x, table):
    b, l = x.shape
    nw = plsc.get_sparse_core_info().num_cores * plsc.get_sparse_core_info().num_subcores
    idx = (x.astype(jnp.int32) * 2).reshape(nw, -1, 128)
    table2 = _transform(table.T).reshape(2 * table.shape[0], _DIM)
    out = _make_gather(b, l, 2 * table.shape[0])(idx, table2)
    t4 = _format(out.reshape(b * l // 2, 2 * _DIM), b, l)
    return t4.transpose(0, 3, 1, 2)


# speedup vs baseline: 1.1814x; 1.0871x over previous
import functools

import jax
import jax.numpy as jnp
from jax import lax
from jax.experimental import pallas as pl
from jax.experimental.pallas import tpu as pltpu
from jax.experimental.pallas import tpu_sc as plsc

_DIM = 64
_SCALE = 8.0
_COLS_BLOCK = 8192


def _transform_body(tt_ref, out_ref):
    pos = lax.broadcasted_iota(jnp.int32, (_COLS_BLOCK, _DIM), 1).astype(jnp.float32) + 1.0
    out_ref[:, : _DIM] = tt_ref[...].T * _SCALE + pos


def _transform(table_t):
    vocab = table_t.shape[1]
    return pl.pallas_call(
        _transform_body,
        grid=((vocab + _COLS_BLOCK - 1) // _COLS_BLOCK,),
        in_specs=[pl.BlockSpec((_DIM, _COLS_BLOCK), lambda i: (0, i))],
        out_specs=pl.BlockSpec((_COLS_BLOCK, 2 * _DIM), lambda i: (i, 0)),
        out_shape=jax.ShapeDtypeStruct((vocab, 2 * _DIM), jnp.float32),
    )(table_t)


_TB = 512


def _format_body(in_ref, out_ref):
    seq2 = in_ref.shape[0] // _TB
    inr = in_ref[...].reshape(_TB, seq2, 2 * _DIM)
    for l2 in range(seq2):
        st = inr[:, l2, :].T
        out_ref[0, 2 * l2] = st[:_DIM]
        out_ref[0, 2 * l2 + 1] = st[_DIM:]


def _format(lin2, batch, seq):
    return pl.pallas_call(
        _format_body,
        grid=(batch // _TB,),
        in_specs=[pl.BlockSpec((_TB * seq // 2, 2 * _DIM), lambda i: (i, 0))],
        out_specs=pl.BlockSpec((1, seq, _DIM, _TB), lambda i: (0, 0, 0, i)),
        out_shape=jax.ShapeDtypeStruct((1, seq, _DIM, batch), jnp.float32),
    )(lin2)


@functools.lru_cache(maxsize=None)
def _make_gather(batch, seq, vocab):
    info = plsc.get_sparse_core_info()
    nc, ns = info.num_cores, info.num_subcores
    nw = nc * ns
    rows = batch * seq
    rpw = rows // nw
    gchunk = 128
    chunk = 256
    nchunks = rpw // chunk
    nbuf = 4
    mesh = plsc.VectorSubcoreMesh(core_axis_name="c", subcore_axis_name="s")

    @functools.partial(
        pl.kernel,
        mesh=mesh,
        compiler_params=pltpu.CompilerParams(use_tc_tiling_on_sc=False),
        out_type=jax.ShapeDtypeStruct((rows, _DIM), jnp.float32),
        scratch_types=[
            pltpu.VMEM((2 * nchunks, gchunk), jnp.int32),
        ]
        + [pltpu.VMEM((chunk, _DIM), jnp.float32)] * nbuf
        + [pltpu.SemaphoreType.DMA] * (2 * nbuf),
    )
    def k(idx_hbm, table_hbm, out_hbm, idx_v, *bufs_sems):
        bufs = bufs_sems[:nbuf]
        gs = bufs_sems[nbuf : 2 * nbuf]
        ws = bufs_sems[2 * nbuf :]
        wid = lax.axis_index("s") * nc + lax.axis_index("c")
        r0 = wid * rpw
        pltpu.sync_copy(idx_hbm.at[wid], idx_v)

        def start_gather(j, p):
            pltpu.async_copy(
                table_hbm.at[idx_v.at[2 * j]], bufs[p].at[pl.ds(0, gchunk)], gs[p])
            pltpu.async_copy(
                table_hbm.at[idx_v.at[2 * j + 1]], bufs[p].at[pl.ds(gchunk, gchunk)],
                gs[p])

        def wait_gather(j, p):
            pltpu.make_async_copy(
                table_hbm.at[idx_v.at[2 * j]], bufs[p].at[pl.ds(0, gchunk)], gs[p]
            ).wait()
            pltpu.make_async_copy(
                table_hbm.at[idx_v.at[2 * j + 1]], bufs[p].at[pl.ds(gchunk, gchunk)],
                gs[p]).wait()

        def start_write(j, p):
            pltpu.async_copy(bufs[p], out_hbm.at[pl.ds(r0 + j * chunk, chunk)], ws[p])

        def wait_write(j, p):
            pltpu.make_async_copy(
                bufs[p], out_hbm.at[pl.ds(r0 + j * chunk, chunk)], ws[p]
            ).wait()

        start_gather(0, 0)
        start_gather(1, 1)

        def body(j4, carry):
            for p in range(nbuf):
                j = nbuf * j4 + p
                wait_gather(j, p)
                start_write(j, p)
                q = (p + 2) % nbuf

                @pl.when(j + 2 < nchunks)
                def _():
                    @pl.when(j >= 2)
                    def _():
                        wait_write(j - 2, q)

                    start_gather(j + 2, q)

            return carry

        lax.fori_loop(0, nchunks // nbuf, body, 0)
        for jt in range(nbuf * (nchunks // nbuf), nchunks):
            wait_gather(jt, jt % nbuf)
            start_write(jt, jt % nbuf)
        for p in range(nbuf):
            j = nchunks - nbuf + p
            wait_write(j, j % nbuf)

    return k


def kernel(x, table):
    b, l = x.shape
    nw = plsc.get_sparse_core_info().num_cores * plsc.get_sparse_core_info().num_subcores
    idx = (x.astype(jnp.int32) * 2).reshape(nw, -1, 128)
    table2 = _transform(table.T).reshape(2 * table.shape[0], _DIM)
    out = _make_gather(b, l, 2 * table.shape[0])(idx, table2)
    t4 = _format(out.reshape(b * l // 2, 2 * _DIM), b, l)
    return t4.transpose(0, 3, 1, 2)
